# baseline (speedup 1.0000x reference)
import jax
import jax.numpy as jnp
from jax.experimental import pallas as pl

K1 = 10
K2 = 20
S = 1.0
L = 100
M = 1000
MP = 1024
LP = 128
QB = 1024
Q = 16384

BIG = 3.0e38
INF = float("inf")
NINF = float("-inf")


NG = MP // 128


def _remove_mins(parts, n_iter):
    assert n_iter % 2 == 0
    lane = jax.lax.broadcasted_iota(jnp.int32, parts[0].shape, 1)

    def body(_, parts):
        m1, m2 = parts[0], jnp.full_like(parts[0], INF)
        for p in parts[1:]:
            m2 = jnp.minimum(m2, jnp.maximum(m1, p))
            m1 = jnp.minimum(m1, p)
        l1 = jnp.argmin(m1, axis=1, keepdims=True)
        v1 = jnp.min(m1, axis=1, keepdims=True)
        cand = jnp.where(lane == l1, m2, m1)
        l2 = jnp.argmin(cand, axis=1, keepdims=True)
        v2 = jnp.min(cand, axis=1, keepdims=True)
        u1 = jnp.where(l1 == l2, v2, v1)
        mval = jnp.where(lane == l1, u1, jnp.where(lane == l2, v2, NINF))
        return tuple(jnp.where(p <= mval, INF, p) for p in parts)

    return jax.lax.fori_loop(0, n_iter // 2, body, parts, unroll=True)


def _split_mask(d):
    parts = [d[:, g * 128:(g + 1) * 128] for g in range(NG)]
    lane = jax.lax.broadcasted_iota(jnp.int32, parts[-1].shape, 1)
    parts[-1] = jnp.where(lane < M - (NG - 1) * 128, parts[-1], BIG)
    return tuple(parts)


def _counts(parts, y):
    n = jnp.concatenate([(p == INF).astype(jnp.float32) for p in parts],
                        axis=1)
    return jnp.dot(n, y, preferred_element_type=jnp.float32)


def _fit_kernel(tf_ref, y_ref, r10_ref, r20_ref):
    tf = tf_ref[...]
    y = y_ref[...]
    tt = jnp.sum(tf * tf, axis=1)
    d = tt[:, None] + tt[None, :] - 2.0 * jnp.dot(
        tf, tf.T, preferred_element_type=jnp.float32)
    parts = _split_mask(d)

    parts = _remove_mins(parts, K1)
    d10 = _counts(parts, y)
    parts = _remove_mins(parts, K2 - K1)
    d20 = _counts(parts, y)

    rowv = (jax.lax.broadcasted_iota(jnp.int32, (MP, 1), 0) < M).astype(
        jnp.float32)
    yv = y * rowv
    cnt1 = jnp.sum(yv, axis=0)
    cnt0 = jnp.float32(M) - cnt1
    prior_t = (S + cnt1) / (2.0 * S + M)
    prior_f = 1.0 - prior_t

    for k, dts, ref, nrows in ((K1, d10, r10_ref, 16), (K2, d20, r20_ref, 24)):
        den_t = S * (k + 1) + cnt1
        den_f = S * (k + 1) + cnt0
        rows = []
        for dd in range(k + 1):
            msk = jnp.where(dts == jnp.float32(dd), 1.0, 0.0) * rowv
            c = jnp.sum(y * msk, axis=0)
            cn = jnp.sum((1.0 - y) * msk, axis=0)
            pt = prior_t * (S + c) / den_t
            pf = prior_f * (S + cn) / den_f
            rows.append(pt / (pt + pf))
        while len(rows) < nrows:
            rows.append(rows[0])
        ref[...] = jnp.stack(rows)


def _pred_kernel(q_ref, tf_ref, y_ref, r10_ref, r20_ref, o10_ref, o20_ref):
    q = q_ref[...]
    tf = tf_ref[...]
    y = y_ref[...]
    tt = jnp.sum(tf * tf, axis=1)
    qq = jnp.sum(q * q, axis=1)
    d = qq[:, None] + tt[None, :] - 2.0 * jnp.dot(
        q, tf.T, preferred_element_type=jnp.float32)
    parts = _split_mask(d)

    parts = _remove_mins(parts, K1)
    d10 = _counts(parts, y)
    parts = _remove_mins(parts, K2 - K1)
    d20 = _counts(parts, y)

    for k, dts, rref, oref in ((K1, d10, r10_ref, o10_ref),
                               (K2, d20, r20_ref, o20_ref)):
        acc = jnp.zeros((QB, LP), jnp.float32)
        for dd in range(k + 1):
            row = rref[dd:dd + 1, :]
            acc = jnp.where(dts == jnp.float32(dd), row, acc)
        oref[...] = acc[:, :L]


def kernel(features, train_features, train_labels):
    f32 = jnp.float32
    tf = jnp.zeros((MP, 64), f32).at[:M].set(train_features.astype(f32))
    y = jnp.zeros((MP, LP), f32).at[:M, :L].set(train_labels.astype(f32))

    r10, r20 = pl.pallas_call(
        _fit_kernel,
        out_shape=[
            jax.ShapeDtypeStruct((16, LP), f32),
            jax.ShapeDtypeStruct((24, LP), f32),
        ],
    )(tf, y)

    o10, o20 = pl.pallas_call(
        _pred_kernel,
        grid=(Q // QB,),
        in_specs=[
            pl.BlockSpec((QB, 64), lambda i: (i, 0)),
            pl.BlockSpec((MP, 64), lambda i: (0, 0)),
            pl.BlockSpec((MP, LP), lambda i: (0, 0)),
            pl.BlockSpec((16, LP), lambda i: (0, 0)),
            pl.BlockSpec((24, LP), lambda i: (0, 0)),
        ],
        out_specs=[
            pl.BlockSpec((QB, L), lambda i: (i, 0)),
            pl.BlockSpec((QB, L), lambda i: (i, 0)),
        ],
        out_shape=[
            jax.ShapeDtypeStruct((Q, L), f32),
            jax.ShapeDtypeStruct((Q, L), f32),
        ],
    )(features.astype(f32), tf, y, r10, r20)

    return (o10, o20)


# speedup vs baseline: 1.0609x; 1.0609x over previous
import jax
import jax.numpy as jnp
from jax.experimental import pallas as pl

K1 = 10
K2 = 20
S = 1.0
L = 100
M = 1000
MP = 1024
LP = 128
QB = 1024
Q = 16384

BIG = 3.0e38
INF = float("inf")
NINF = float("-inf")


NG = MP // 128


def _remove_mins(parts, n_iter):
    lane = jax.lax.broadcasted_iota(jnp.int32, parts[0].shape, 1)

    def body(_, parts):
        comb = parts[0]
        for p in parts[1:]:
            comb = jnp.minimum(comb, p)
        l = jnp.argmin(comb, axis=1, keepdims=True)
        m = jnp.min(comb, axis=1, keepdims=True)
        mval = jnp.where(lane == l, m, NINF)
        return tuple(jnp.where(p == mval, INF, p) for p in parts)

    return jax.lax.fori_loop(0, n_iter, body, parts, unroll=True)


def _split_mask(d):
    parts = [d[:, g * 128:(g + 1) * 128] for g in range(NG)]
    lane = jax.lax.broadcasted_iota(jnp.int32, parts[-1].shape, 1)
    parts[-1] = jnp.where(lane < M - (NG - 1) * 128, parts[-1], BIG)
    return tuple(parts)


def _counts(parts, y):
    n = jnp.concatenate([(p == INF).astype(jnp.float32) for p in parts],
                        axis=1)
    return jnp.dot(n, y, preferred_element_type=jnp.float32)


def _fit_kernel(tf_ref, y_ref, r10_ref, r20_ref):
    tf = tf_ref[...]
    y = y_ref[...]
    tt = jnp.sum(tf * tf, axis=1)
    d = tt[:, None] + tt[None, :] - 2.0 * jnp.dot(
        tf, tf.T, preferred_element_type=jnp.float32)
    parts = _split_mask(d)

    parts = _remove_mins(parts, K1)
    d10 = _counts(parts, y)
    parts = _remove_mins(parts, K2 - K1)
    d20 = _counts(parts, y)

    rowv = (jax.lax.broadcasted_iota(jnp.int32, (MP, 1), 0) < M).astype(
        jnp.float32)
    yv = y * rowv
    ones = jnp.ones((8, MP), jnp.float32)
    cnt1 = jnp.dot(ones, yv, preferred_element_type=jnp.float32)[0]
    cnt0 = jnp.float32(M) - cnt1
    prior_t = (S + cnt1) / (2.0 * S + M)
    prior_f = 1.0 - prior_t

    for k, dts, ref, nrows in ((K1, d10, r10_ref, 16), (K2, d20, r20_ref, 24)):
        dts_m = jnp.where(rowv > 0.0, dts, -1.0)
        den_t = S * (k + 1) + cnt1
        den_f = S * (k + 1) + cnt0
        blocks = []
        for dd in range(k + 1):
            msk = jnp.where(dts_m == jnp.float32(dd), 1.0, 0.0)
            blocks.append(yv * msk)
            blocks.append(msk)
        sums = jnp.dot(ones, jnp.concatenate(blocks, axis=1),
                       preferred_element_type=jnp.float32)[0]
        rows = []
        for dd in range(k + 1):
            c = sums[2 * dd * LP:(2 * dd + 1) * LP]
            cnt_dd = sums[(2 * dd + 1) * LP:(2 * dd + 2) * LP]
            pt = prior_t * (S + c) / den_t
            pf = prior_f * (S + (cnt_dd - c)) / den_f
            rows.append(pt / (pt + pf))
        while len(rows) < nrows:
            rows.append(rows[0])
        ref[...] = jnp.stack(rows)


def _pred_kernel(q_ref, tf_ref, y_ref, r10_ref, r20_ref, o10_ref, o20_ref):
    q = q_ref[...]
    tf = tf_ref[...]
    y = y_ref[...]
    tt = jnp.sum(tf * tf, axis=1)
    qq = jnp.sum(q * q, axis=1)
    d = qq[:, None] + tt[None, :] - 2.0 * jnp.dot(
        q, tf.T, preferred_element_type=jnp.float32)
    parts = _split_mask(d)

    parts = _remove_mins(parts, K1)
    d10 = _counts(parts, y)
    parts = _remove_mins(parts, K2 - K1)
    d20 = _counts(parts, y)

    for k, dts, rref, oref in ((K1, d10, r10_ref, o10_ref),
                               (K2, d20, r20_ref, o20_ref)):
        acc = jnp.zeros((QB, LP), jnp.float32)
        for dd in range(k + 1):
            row = rref[dd:dd + 1, :]
            acc = jnp.where(dts == jnp.float32(dd), row, acc)
        oref[...] = acc[:, :L]


def kernel(features, train_features, train_labels):
    f32 = jnp.float32
    tf = jnp.zeros((MP, 64), f32).at[:M].set(train_features.astype(f32))
    y = jnp.zeros((MP, LP), f32).at[:M, :L].set(train_labels.astype(f32))

    r10, r20 = pl.pallas_call(
        _fit_kernel,
        out_shape=[
            jax.ShapeDtypeStruct((16, LP), f32),
            jax.ShapeDtypeStruct((24, LP), f32),
        ],
    )(tf, y)

    o10, o20 = pl.pallas_call(
        _pred_kernel,
        grid=(Q // QB,),
        in_specs=[
            pl.BlockSpec((QB, 64), lambda i: (i, 0)),
            pl.BlockSpec((MP, 64), lambda i: (0, 0)),
            pl.BlockSpec((MP, LP), lambda i: (0, 0)),
            pl.BlockSpec((16, LP), lambda i: (0, 0)),
            pl.BlockSpec((24, LP), lambda i: (0, 0)),
        ],
        out_specs=[
            pl.BlockSpec((QB, L), lambda i: (i, 0)),
            pl.BlockSpec((QB, L), lambda i: (i, 0)),
        ],
        out_shape=[
            jax.ShapeDtypeStruct((Q, L), f32),
            jax.ShapeDtypeStruct((Q, L), f32),
        ],
    )(features.astype(f32), tf, y, r10, r20)

    return (o10, o20)
